# baseline (speedup 1.0000x reference)
import functools

import jax
import jax.numpy as jnp
from jax import lax
from jax.experimental import pallas as pl
from jax.experimental.pallas import tpu as pltpu
from jax.experimental.pallas import tpu_sc as plsc

N = 10000
NP = 10240
E = 160000
IN_CH = 256
HID = 512
OUT_CH = 256

NC = 2
NS = 16
K = 128
EPS = 10240
NB = EPS // K
PE = NS * EPS
DUMP = NP - 1
RPS = NP // NS

RB = 1024
GRID = NP // RB

_mesh = lambda: plsc.VectorSubcoreMesh(core_axis_name="c", subcore_axis_name="s")


def _deg_pass(dst3, ones_t, zeros_d):
    @functools.partial(
        pl.kernel,
        out_type=jax.ShapeDtypeStruct((NC, NP, 128), jnp.float32),
        mesh=_mesh(),
        scratch_types=[
            pltpu.VMEM((NB, K), jnp.int32),
            pltpu.VMEM((K, 128), jnp.float32),
            pltpu.VMEM_SHARED((NP, 128), jnp.float32),
        ],
    )
    def deg_kernel(dst_hbm, ones_hbm, zeros_hbm, deg_out, dst_v, ones_v, deg_sh):
        core = lax.axis_index("c")
        s = lax.axis_index("s")
        pltpu.sync_copy(dst_hbm.at[s], dst_v)
        pltpu.sync_copy(ones_hbm, ones_v)
        pltpu.sync_copy(zeros_hbm.at[pl.ds(s * RPS, RPS)],
                        deg_sh.at[pl.ds(s * RPS, RPS)])
        plsc.subcore_barrier()

        half = NB // NC

        @pl.loop(0, half)
        def _(j):
            pltpu.sync_copy(ones_v, deg_sh.at[dst_v.at[core * half + j]],
                            add=True)

        plsc.subcore_barrier()
        pltpu.sync_copy(deg_sh.at[pl.ds(s * RPS, RPS)],
                        deg_out.at[core, pl.ds(s * RPS, RPS)])

    return deg_kernel(dst3, ones_t, zeros_d)


def _sc_pass(h_flat, srcC, dst3, n_chunks):
    cp = n_chunks // NC
    KS = 64
    NBS = NB * 2
    QB = NBS // 4
    D = 4

    @functools.partial(
        pl.kernel,
        out_type=jax.ShapeDtypeStruct((n_chunks, NP, 128), jnp.float32),
        mesh=_mesh(),
        scratch_types=[
            pltpu.VMEM((NB, K), jnp.int32),
            pltpu.VMEM((QB, KS), jnp.int32),
        ]
        + [pltpu.VMEM((KS, 128), jnp.float32) for _ in range(D)]
        + [pltpu.SemaphoreType.DMA for _ in range(2 * D)]
        + [pltpu.VMEM_SHARED((NP, 128), jnp.float32)],
    )
    def sc_kernel(h_hbm, src_hbm, dst_hbm, acc_out, src_v, dst_v, *rest):
        rows = rest[:D]
        gsem = rest[D:2 * D]
        ssem = rest[2 * D:3 * D]
        acc_sh = rest[3 * D]
        core = lax.axis_index("c")
        s = lax.axis_index("s")

        def src_slice(j, p):
            return src_v.at[j // 2, pl.ds(p * KS, KS)]

        def gather(j, b):
            pltpu.async_copy(h_hbm.at[src_slice(j, j % 2)], rows[b], gsem[b])

        def wait_gather(j, b):
            pltpu.make_async_copy(h_hbm.at[src_slice(j, j % 2)], rows[b],
                                  gsem[b]).wait()

        def wait_scatter(b):
            pltpu.make_async_copy(rows[b], acc_sh.at[dst_v.at[0]],
                                  ssem[b]).wait()

        for i in range(cp):
            ch = core * cp + i
            pltpu.sync_copy(src_hbm.at[ch, s], src_v)
            pltpu.sync_copy(h_hbm.at[pl.ds(ch * NP + s * RPS, RPS)],
                            acc_sh.at[pl.ds(s * RPS, RPS)])
            plsc.subcore_barrier()

            gather(0, 0)
            gather(1, 1)
            for q in range(4):
                pltpu.sync_copy(dst_hbm.at[s, pl.ds(q * QB, QB)], dst_v)

                @pl.loop(0, QB // D)
                def _(g):
                    for b0 in range(D):
                        jq = g * D + b0
                        j = q * QB + jq
                        bn = (b0 + 2) % D
                        if b0 >= 2:
                            wait_scatter(bn)

                            @pl.when(j + 2 < NBS)
                            def _():
                                gather(j + 2, bn)
                        else:
                            @pl.when(g >= 1)
                            def _():
                                wait_scatter(bn)

                            @pl.when(j + 2 < NBS)
                            def _():
                                gather(j + 2, bn)
                        wait_gather(j, b0)
                        pltpu.async_copy(rows[b0], acc_sh.at[dst_v.at[jq]],
                                         ssem[b0], add=True)

                wait_scatter(2)
                wait_scatter(3)

            plsc.subcore_barrier()
            pltpu.sync_copy(acc_sh.at[pl.ds(s * RPS, RPS)],
                            acc_out.at[ch, pl.ds(s * RPS, RPS)])
            plsc.subcore_barrier()

    return sc_kernel(h_flat, srcC, dst3)


def _tc_first(degp, x, w1):
    def body(degp_ref, x_ref, w_ref, dis_ref, h_ref):
        degp = degp_ref[:, :, 0:1]
        deg = degp[0] + degp[1] + 1.0

        dis = lax.rsqrt(deg)
        dis_ref[...] = dis
        h = jnp.dot(x_ref[...], w_ref[...],
                    preferred_element_type=jnp.float32,
                    precision=lax.Precision.HIGHEST) * dis
        for c in range(HID // 128):
            h_ref[c] = h[:, c * 128:(c + 1) * 128]

    return pl.pallas_call(
        body,
        grid=(GRID,),
        in_specs=[
            pl.BlockSpec((NC, RB, 128), lambda r: (0, r, 0)),
            pl.BlockSpec((RB, IN_CH), lambda r: (r, 0)),
            pl.BlockSpec((IN_CH, HID), lambda r: (0, 0)),
        ],
        out_specs=[
            pl.BlockSpec((RB, 1), lambda r: (r, 0)),
            pl.BlockSpec((HID // 128, RB, 128), lambda r: (0, r, 0)),
        ],
        out_shape=[
            jax.ShapeDtypeStruct((NP, 1), jnp.float32),
            jax.ShapeDtypeStruct((HID // 128, NP, 128), jnp.float32),
        ],
    )(degp, x, w1)


def _tc_mid(acc, dis, b, w, c_out):
    c_in = acc.shape[0]
    k_dim = c_in * 128

    def body(acc_ref, dis_ref, b_ref, w_ref, h_ref):
        dis = dis_ref[...]
        zs = [
            jnp.maximum(acc_ref[c] * dis + b_ref[0:1, c * 128:(c + 1) * 128],
                        0.0)
            for c in range(c_in)
        ]
        z = jnp.concatenate(zs, axis=1)
        h = jnp.dot(z, w_ref[...],
                    preferred_element_type=jnp.float32,
                    precision=lax.Precision.HIGHEST) * dis
        for c in range(c_out):
            h_ref[c] = h[:, c * 128:(c + 1) * 128]

    return pl.pallas_call(
        body,
        grid=(GRID,),
        in_specs=[
            pl.BlockSpec((c_in, RB, 128), lambda r: (0, r, 0)),
            pl.BlockSpec((RB, 1), lambda r: (r, 0)),
            pl.BlockSpec((1, k_dim), lambda r: (0, 0)),
            pl.BlockSpec((k_dim, c_out * 128), lambda r: (0, 0)),
        ],
        out_specs=pl.BlockSpec((c_out, RB, 128), lambda r: (0, r, 0)),
        out_shape=jax.ShapeDtypeStruct((c_out, NP, 128), jnp.float32),
    )(acc, dis, b, w)


def _tc_final(acc, dis, b):
    c_in = acc.shape[0]
    rb = 1000

    def body(acc_ref, dis_ref, b_ref, o_ref):
        dis = dis_ref[...]
        for c in range(c_in):
            o_ref[:, c * 128:(c + 1) * 128] = (
                acc_ref[c] * dis + b_ref[0:1, c * 128:(c + 1) * 128])

    return pl.pallas_call(
        body,
        grid=(N // rb,),
        in_specs=[
            pl.BlockSpec((c_in, rb, 128), lambda r: (0, r, 0)),
            pl.BlockSpec((rb, 1), lambda r: (r, 0)),
            pl.BlockSpec((1, OUT_CH), lambda r: (0, 0)),
        ],
        out_specs=pl.BlockSpec((rb, OUT_CH), lambda r: (r, 0)),
        out_shape=jax.ShapeDtypeStruct((N, OUT_CH), jnp.float32),
    )(acc, dis, b)


def kernel(x, edge_index, W1, b1, W2, b2, W3, b3):
    src = edge_index[0].astype(jnp.int32)
    dst = edge_index[1].astype(jnp.int32)
    pad = PE - E
    src_p = jnp.concatenate([src, jnp.zeros((pad,), jnp.int32)])
    dst_p = jnp.concatenate([dst, jnp.full((pad,), DUMP, jnp.int32)])
    src3 = src_p.reshape(NS, NB, K)
    dst3 = dst_p.reshape(NS, NB, K)
    dst3s = dst_p.reshape(NS, NB * 2, 64)
    offs = (jnp.arange(4, dtype=jnp.int32) * NP)[:, None, None, None]
    src4 = src3[None] + offs
    src2 = src4[:2]
    ones_t = jnp.ones((K, 128), jnp.float32)
    zeros_d = jnp.zeros((NP, 128), jnp.float32)
    x_pad = jnp.zeros((NP, IN_CH), jnp.float32).at[:N].set(x)
    b1r = b1.reshape(1, HID)
    b2r = b2.reshape(1, HID)
    b3r = b3.reshape(1, OUT_CH)

    degp = _deg_pass(dst3, ones_t, zeros_d)
    dis, h1 = _tc_first(degp, x_pad, W1)
    acc1 = _sc_pass(h1.reshape(4 * NP, 128), src4, dst3s, 4)
    h2 = _tc_mid(acc1, dis, b1r, W2, 4)
    acc2 = _sc_pass(h2.reshape(4 * NP, 128), src4, dst3s, 4)
    h3 = _tc_mid(acc2, dis, b2r, W3, 2)
    acc3 = _sc_pass(h3.reshape(2 * NP, 128), src2, dst3s, 2)
    return _tc_final(acc3, dis, b3r)


# speedup vs baseline: 1.0775x; 1.0775x over previous
import functools

import jax
import jax.numpy as jnp
from jax import lax
from jax.experimental import pallas as pl
from jax.experimental.pallas import tpu as pltpu
from jax.experimental.pallas import tpu_sc as plsc

N = 10000
NP = 10240
E = 160000
IN_CH = 256
HID = 512
OUT_CH = 256

NC = 2
NS = 16
K = 128
EPS = 10240
NB = EPS // K
PE = NS * EPS
DUMP = NP - 1
RPS = NP // NS

RB = 1024
GRID = NP // RB

_mesh = lambda: plsc.VectorSubcoreMesh(core_axis_name="c", subcore_axis_name="s")


def _deg_pass(dst3, ones_t, zeros_d):
    @functools.partial(
        pl.kernel,
        out_type=jax.ShapeDtypeStruct((NC, NP, 128), jnp.float32),
        mesh=_mesh(),
        scratch_types=[
            pltpu.VMEM((NB, K), jnp.int32),
            pltpu.VMEM((K, 128), jnp.float32),
            pltpu.VMEM_SHARED((NP, 128), jnp.float32),
        ],
    )
    def deg_kernel(dst_hbm, ones_hbm, zeros_hbm, deg_out, dst_v, ones_v, deg_sh):
        core = lax.axis_index("c")
        s = lax.axis_index("s")
        pltpu.sync_copy(dst_hbm.at[s], dst_v)
        pltpu.sync_copy(ones_hbm, ones_v)
        pltpu.sync_copy(zeros_hbm.at[pl.ds(s * RPS, RPS)],
                        deg_sh.at[pl.ds(s * RPS, RPS)])
        plsc.subcore_barrier()

        half = NB // NC

        @pl.loop(0, half)
        def _(j):
            pltpu.sync_copy(ones_v, deg_sh.at[dst_v.at[core * half + j]],
                            add=True)

        plsc.subcore_barrier()
        pltpu.sync_copy(deg_sh.at[pl.ds(s * RPS, RPS)],
                        deg_out.at[core, pl.ds(s * RPS, RPS)])

    return deg_kernel(dst3, ones_t, zeros_d)


def _sc_pass(h_flat, srcC, dst3, n_chunks):
    cp = n_chunks // NC
    HB = NB // 2

    @functools.partial(
        pl.kernel,
        out_type=jax.ShapeDtypeStruct((n_chunks, NP, 128), jnp.float32),
        mesh=_mesh(),
        scratch_types=[
            pltpu.VMEM((NB, K), jnp.int32),
            pltpu.VMEM((HB, K), jnp.int32),
            pltpu.VMEM((K, 128), jnp.float32),
            pltpu.VMEM((K, 128), jnp.float32),
            pltpu.SemaphoreType.DMA,
            pltpu.SemaphoreType.DMA,
            pltpu.VMEM_SHARED((NP, 128), jnp.float32),
        ],
    )
    def sc_kernel(h_hbm, src_hbm, dst_hbm, acc_out, src_v, dst_v,
                  rows0, rows1, sem0, sem1, acc_sh):
        rows = (rows0, rows1)
        sems = (sem0, sem1)
        core = lax.axis_index("c")
        s = lax.axis_index("s")
        for i in range(cp):
            ch = core * cp + i
            pltpu.sync_copy(src_hbm.at[ch, s], src_v)
            pltpu.sync_copy(h_hbm.at[pl.ds(ch * NP + s * RPS, RPS)],
                            acc_sh.at[pl.ds(s * RPS, RPS)])
            plsc.subcore_barrier()

            pltpu.async_copy(h_hbm.at[src_v.at[0]], rows[0], sems[0])
            pltpu.async_copy(h_hbm.at[src_v.at[1]], rows[1], sems[1])
            for half in range(2):
                pltpu.sync_copy(dst_hbm.at[s, pl.ds(half * HB, HB)], dst_v)

                @pl.loop(0, HB // 2)
                def _(g):
                    for b in range(2):
                        jl = g * 2 + b
                        j = half * HB + jl
                        pltpu.make_async_copy(h_hbm.at[src_v.at[j]], rows[b],
                                              sems[b]).wait()
                        pltpu.sync_copy(rows[b], acc_sh.at[dst_v.at[jl]],
                                        add=True)

                        @pl.when(j + 2 < NB)
                        def _():
                            pltpu.async_copy(h_hbm.at[src_v.at[j + 2]],
                                             rows[b], sems[b])

            plsc.subcore_barrier()
            pltpu.sync_copy(acc_sh.at[pl.ds(s * RPS, RPS)],
                            acc_out.at[ch, pl.ds(s * RPS, RPS)])
            plsc.subcore_barrier()

    return sc_kernel(h_flat, srcC, dst3)


def _tc_first(degp, x, w1):
    def body(degp_ref, x_ref, w_ref, dis_ref, h_ref):
        degp = degp_ref[:, :, 0:1]
        deg = degp[0] + degp[1] + 1.0

        dis = lax.rsqrt(deg)
        dis_ref[...] = dis
        h = jnp.dot(x_ref[...], w_ref[...],
                    preferred_element_type=jnp.float32,
                    precision=lax.Precision.HIGHEST) * dis
        for c in range(HID // 128):
            h_ref[c] = h[:, c * 128:(c + 1) * 128]

    return pl.pallas_call(
        body,
        grid=(GRID,),
        in_specs=[
            pl.BlockSpec((NC, RB, 128), lambda r: (0, r, 0)),
            pl.BlockSpec((RB, IN_CH), lambda r: (r, 0)),
            pl.BlockSpec((IN_CH, HID), lambda r: (0, 0)),
        ],
        out_specs=[
            pl.BlockSpec((RB, 1), lambda r: (r, 0)),
            pl.BlockSpec((HID // 128, RB, 128), lambda r: (0, r, 0)),
        ],
        out_shape=[
            jax.ShapeDtypeStruct((NP, 1), jnp.float32),
            jax.ShapeDtypeStruct((HID // 128, NP, 128), jnp.float32),
        ],
    )(degp, x, w1)


def _tc_mid(acc, dis, b, w, c_out):
    c_in = acc.shape[0]
    k_dim = c_in * 128

    def body(acc_ref, dis_ref, b_ref, w_ref, h_ref):
        dis = dis_ref[...]
        zs = [
            jnp.maximum(acc_ref[c] * dis + b_ref[0:1, c * 128:(c + 1) * 128],
                        0.0)
            for c in range(c_in)
        ]
        z = jnp.concatenate(zs, axis=1)
        h = jnp.dot(z, w_ref[...],
                    preferred_element_type=jnp.float32,
                    precision=lax.Precision.HIGHEST) * dis
        for c in range(c_out):
            h_ref[c] = h[:, c * 128:(c + 1) * 128]

    return pl.pallas_call(
        body,
        grid=(GRID,),
        in_specs=[
            pl.BlockSpec((c_in, RB, 128), lambda r: (0, r, 0)),
            pl.BlockSpec((RB, 1), lambda r: (r, 0)),
            pl.BlockSpec((1, k_dim), lambda r: (0, 0)),
            pl.BlockSpec((k_dim, c_out * 128), lambda r: (0, 0)),
        ],
        out_specs=pl.BlockSpec((c_out, RB, 128), lambda r: (0, r, 0)),
        out_shape=jax.ShapeDtypeStruct((c_out, NP, 128), jnp.float32),
    )(acc, dis, b, w)


def _tc_final(acc, dis, b):
    c_in = acc.shape[0]
    rb = 1000

    def body(acc_ref, dis_ref, b_ref, o_ref):
        dis = dis_ref[...]
        for c in range(c_in):
            o_ref[:, c * 128:(c + 1) * 128] = (
                acc_ref[c] * dis + b_ref[0:1, c * 128:(c + 1) * 128])

    return pl.pallas_call(
        body,
        grid=(N // rb,),
        in_specs=[
            pl.BlockSpec((c_in, rb, 128), lambda r: (0, r, 0)),
            pl.BlockSpec((rb, 1), lambda r: (r, 0)),
            pl.BlockSpec((1, OUT_CH), lambda r: (0, 0)),
        ],
        out_specs=pl.BlockSpec((rb, OUT_CH), lambda r: (r, 0)),
        out_shape=jax.ShapeDtypeStruct((N, OUT_CH), jnp.float32),
    )(acc, dis, b)


def kernel(x, edge_index, W1, b1, W2, b2, W3, b3):
    src = edge_index[0].astype(jnp.int32)
    dst = edge_index[1].astype(jnp.int32)
    pad = PE - E
    src_p = jnp.concatenate([src, jnp.zeros((pad,), jnp.int32)])
    dst_p = jnp.concatenate([dst, jnp.full((pad,), DUMP, jnp.int32)])
    src3 = src_p.reshape(NS, NB, K)
    dst3 = dst_p.reshape(NS, NB, K)
    offs = (jnp.arange(4, dtype=jnp.int32) * NP)[:, None, None, None]
    src4 = src3[None] + offs
    src2 = src4[:2]
    ones_t = jnp.ones((K, 128), jnp.float32)
    zeros_d = jnp.zeros((NP, 128), jnp.float32)
    x_pad = jnp.zeros((NP, IN_CH), jnp.float32).at[:N].set(x)
    b1r = b1.reshape(1, HID)
    b2r = b2.reshape(1, HID)
    b3r = b3.reshape(1, OUT_CH)

    degp = _deg_pass(dst3, ones_t, zeros_d)
    dis, h1 = _tc_first(degp, x_pad, W1)
    acc1 = _sc_pass(h1.reshape(4 * NP, 128), src4, dst3, 4)
    h2 = _tc_mid(acc1, dis, b1r, W2, 4)
    acc2 = _sc_pass(h2.reshape(4 * NP, 128), src4, dst3, 4)
    h3 = _tc_mid(acc2, dis, b2r, W3, 2)
    acc3 = _sc_pass(h3.reshape(2 * NP, 128), src2, dst3, 2)
    return _tc_final(acc3, dis, b3r)
